# baseline (speedup 1.0000x reference)
import functools
import jax
import jax.numpy as jnp
from jax import lax
from jax.experimental import pallas as pl
from jax.experimental.pallas import tpu as pltpu
from jax.experimental.pallas import tpu_sc as plsc

BATCH = 16384
NUM_CORES = 2
NUM_SUBCORES = 16
LANES = 16
NUM_WORKERS = NUM_CORES * NUM_SUBCORES
B_PER_W = BATCH // NUM_WORKERS


def _build():
    mesh = plsc.VectorSubcoreMesh(core_axis_name="c", subcore_axis_name="s")

    @functools.partial(
        pl.kernel,
        mesh=mesh,
        out_type=jax.ShapeDtypeStruct((BATCH,), jnp.float32),
        scratch_types=[
            pltpu.VMEM((B_PER_W,), jnp.int32),
            pltpu.VMEM((B_PER_W,), jnp.int32),
            pltpu.VMEM((B_PER_W,), jnp.float32),
            pltpu.VMEM((B_PER_W,), jnp.float32),
            pltpu.VMEM((LANES,), jnp.float32),
            pltpu.SemaphoreType.DMA,
            pltpu.SemaphoreType.DMA,
        ],
    )
    def gather_sum(uid_hbm, vid_hbm, bu_hbm, bv_hbm, bg_hbm, out_hbm,
                   uidx_v, vidx_v, u1_v, v1_v, bg_v, sem_u, sem_v):
        wid = lax.axis_index("s") * NUM_CORES + lax.axis_index("c")
        base = wid * B_PER_W
        pltpu.sync_copy(uid_hbm.at[pl.ds(base, B_PER_W)], uidx_v)
        pltpu.sync_copy(vid_hbm.at[pl.ds(base, B_PER_W)], vidx_v)
        cu = pltpu.async_copy(bu_hbm.at[uidx_v], u1_v, sem_u)
        cv = pltpu.async_copy(bv_hbm.at[vidx_v], v1_v, sem_v)
        pltpu.sync_copy(bg_hbm, bg_v)
        cu.wait()
        cv.wait()
        bg = bg_v[...]
        for i in range(B_PER_W // LANES):
            sl = pl.ds(i * LANES, LANES)
            u1_v[sl] = u1_v[sl] + v1_v[sl] + bg
        pltpu.sync_copy(u1_v, out_hbm.at[pl.ds(base, B_PER_W)])

    return gather_sum


_gather_sum = _build()


@jax.jit
def kernel(uid, vid, b_u, b_v, b_g):
    bu = jnp.reshape(b_u, (-1,))
    bv = jnp.reshape(b_v, (-1,))
    bg16 = jnp.broadcast_to(b_g.astype(jnp.float32), (LANES,))
    out = _gather_sum(uid.astype(jnp.int32), vid.astype(jnp.int32),
                      bu, bv, bg16)
    return jnp.reshape(out, (-1, 1))


# speedup vs baseline: 4.8810x; 4.8810x over previous
import functools
import jax
import jax.numpy as jnp
from jax import lax
from jax.experimental import pallas as pl
from jax.experimental.pallas import tpu as pltpu
from jax.experimental.pallas import tpu_sc as plsc

BATCH = 16384
NUM_CORES = 2
NUM_SUBCORES = 16
LANES = 16
NUM_WORKERS = NUM_CORES * NUM_SUBCORES
B_PER_W = BATCH // NUM_WORKERS


def _build():
    mesh = plsc.VectorSubcoreMesh(core_axis_name="c", subcore_axis_name="s")

    @functools.partial(
        pl.kernel,
        mesh=mesh,
        out_type=jax.ShapeDtypeStruct((BATCH,), jnp.float32),
        scratch_types=[
            pltpu.VMEM((B_PER_W,), jnp.int32),
            pltpu.VMEM((B_PER_W,), jnp.int32),
            pltpu.VMEM((B_PER_W,), jnp.float32),
            pltpu.VMEM((B_PER_W,), jnp.float32),
            pltpu.VMEM((LANES,), jnp.float32),
            pltpu.SemaphoreType.DMA,
            pltpu.SemaphoreType.DMA,
        ],
    )
    def gather_sum(uid_hbm, vid_hbm, bu_hbm, bv_hbm, bg_hbm, out_hbm,
                   uidx_v, vidx_v, u1_v, v1_v, bg_v, sem_u, sem_v):
        wid = lax.axis_index("s") * NUM_CORES + lax.axis_index("c")
        base = wid * B_PER_W
        pltpu.sync_copy(uid_hbm.at[pl.ds(base, B_PER_W)], uidx_v)
        pltpu.sync_copy(vid_hbm.at[pl.ds(base, B_PER_W)], vidx_v)
        cu = pltpu.async_copy(bu_hbm.at[0].at[uidx_v], u1_v, sem_u)
        cv = pltpu.async_copy(bv_hbm.at[0].at[vidx_v], v1_v, sem_v)
        pltpu.sync_copy(bg_hbm, bg_v)
        cu.wait()
        cv.wait()
        bg = bg_v[...]
        for i in range(B_PER_W // LANES):
            sl = pl.ds(i * LANES, LANES)
            u1_v[sl] = u1_v[sl] + v1_v[sl] + bg
        pltpu.sync_copy(u1_v, out_hbm.at[pl.ds(base, B_PER_W)])

    return gather_sum


_gather_sum = _build()


@jax.jit
def kernel(uid, vid, b_u, b_v, b_g):
    bu = jnp.transpose(b_u)
    bv = jnp.transpose(b_v)
    bg16 = jnp.broadcast_to(b_g.astype(jnp.float32), (LANES,))
    out = _gather_sum(uid.astype(jnp.int32), vid.astype(jnp.int32),
                      bu, bv, bg16)
    return jnp.reshape(out, (-1, 1))


# speedup vs baseline: 4.9074x; 1.0054x over previous
import functools
import jax
import jax.numpy as jnp
from jax import lax
from jax.experimental import pallas as pl
from jax.experimental.pallas import tpu as pltpu
from jax.experimental.pallas import tpu_sc as plsc

BATCH = 16384
NUM_CORES = 2
NUM_SUBCORES = 16
LANES = 16
NUM_WORKERS = NUM_CORES * NUM_SUBCORES
B_PER_W = BATCH // NUM_WORKERS


def _build():
    mesh = plsc.VectorSubcoreMesh(core_axis_name="c", subcore_axis_name="s")

    @functools.partial(
        pl.kernel,
        mesh=mesh,
        out_type=jax.ShapeDtypeStruct((BATCH,), jnp.float32),
        scratch_types=[
            pltpu.VMEM((B_PER_W,), jnp.int32),
            pltpu.VMEM((B_PER_W,), jnp.int32),
            pltpu.VMEM((B_PER_W,), jnp.float32),
            pltpu.VMEM((B_PER_W,), jnp.float32),
            pltpu.VMEM((LANES,), jnp.float32),
            pltpu.SemaphoreType.DMA,
            pltpu.SemaphoreType.DMA,
            pltpu.SemaphoreType.DMA,
        ],
    )
    def gather_sum(uid_hbm, vid_hbm, bu_hbm, bv_hbm, bg_hbm, out_hbm,
                   uidx_v, vidx_v, u1_v, v1_v, bg_v, sem_u, sem_v, sem_i):
        wid = lax.axis_index("s") * NUM_CORES + lax.axis_index("c")
        base = wid * B_PER_W
        ci = pltpu.async_copy(uid_hbm.at[pl.ds(base, B_PER_W)], uidx_v,
                              sem_i)
        cj = pltpu.async_copy(vid_hbm.at[pl.ds(base, B_PER_W)], vidx_v,
                              sem_i)
        pltpu.sync_copy(bg_hbm, bg_v)
        ci.wait()
        cj.wait()
        cu = pltpu.async_copy(bu_hbm.at[0].at[uidx_v], u1_v, sem_u)
        cv = pltpu.async_copy(bv_hbm.at[0].at[vidx_v], v1_v, sem_v)
        bg = bg_v[...]
        cu.wait()
        cv.wait()
        for i in range(B_PER_W // LANES):
            sl = pl.ds(i * LANES, LANES)
            u1_v[sl] = u1_v[sl] + v1_v[sl] + bg
        pltpu.sync_copy(u1_v, out_hbm.at[pl.ds(base, B_PER_W)])

    return gather_sum


_gather_sum = _build()


@jax.jit
def kernel(uid, vid, b_u, b_v, b_g):
    bu = jnp.transpose(b_u)
    bv = jnp.transpose(b_v)
    bg16 = jnp.broadcast_to(b_g.astype(jnp.float32), (LANES,))
    out = _gather_sum(uid.astype(jnp.int32), vid.astype(jnp.int32),
                      bu, bv, bg16)
    return jnp.reshape(out, (-1, 1))
